# Initial kernel scaffold; baseline (speedup 1.0000x reference)
import jax
import jax.numpy as jnp
from jax.experimental import pallas as pl


def kernel(query, reference_points, value, spatial_shapes, level_start_index, Wv, bv, W_off, b_off, W_att, b_att, Wo, bo):
    raise NotImplementedError("write your pallas kernel here")


# speedup vs baseline: 56.7578x; 56.7578x over previous
import functools

import numpy as np
import jax
import jax.numpy as jnp
from jax import lax
from jax.experimental import pallas as pl
from jax.experimental.pallas import tpu as pltpu
from jax.experimental.pallas import tpu_sc as plsc

Q = 16384
CDIM = 256
NH = 8
NN = 2
NP = 4
GRID = 128
D = 32
NTAB = NN * NH * Q

BQ = 1024
NW = 32
QC = Q // NW
SUP = 32
SQ = 4
NSTEP = SUP // SQ
NSUP = QC // SUP

_f32 = jnp.float32
_i32 = jnp.int32

_H_OF_LANE = np.arange(64) // 8
_N_OF_LANE = (np.arange(64) // 4) % 2
_BASE_LANE = ((_N_OF_LANE * NH + _H_OF_LANE) * Q).astype(np.float32).reshape(1, 64)
_GG = np.kron(np.eye(16, dtype=np.float32), np.ones((4, 4), np.float32))
_PERM = np.zeros((256, 256), np.float32)
for _c in range(4):
    for _hn in range(16):
        for _p in range(4):
            _PERM[_c * 64 + _hn * 4 + _p, _hn * 16 + _p * 4 + _c] = 1.0


def _tc1_body(q_ref, val_ref, rpx_ref, rpy_ref, wv_ref, bv_ref, wtop_ref,
              wbot_ref, bcat_ref, gg_ref, base_ref, perm_ref,
              vtab_ref, idx_ref, wts_ref):
    qb = q_ref[0]
    v0 = val_ref[0]
    v1 = val_ref[1]
    vv0 = jnp.dot(v0, wv_ref[...], preferred_element_type=_f32) + bv_ref[...]
    vv1 = jnp.dot(v1, wv_ref[...], preferred_element_type=_f32) + bv_ref[...]
    for h in range(NH):
        vtab_ref[0, h] = vv0[:, h * D:(h + 1) * D]
        vtab_ref[1, h] = vv1[:, h * D:(h + 1) * D]
    t = (jnp.dot(v0, wtop_ref[...], preferred_element_type=_f32)
         + jnp.dot(qb, wbot_ref[...], preferred_element_type=_f32)
         + bcat_ref[...])
    offx = t[:, 0:64]
    offy = t[:, 64:128]
    al = t[:, 128:192]
    e = jnp.exp(al - jnp.max(al, axis=-1, keepdims=True))
    den = jnp.dot(e, gg_ref[...], preferred_element_type=_f32)
    attw = e / den
    x = rpx_ref[...] * float(GRID) - 0.5 + offx
    y = rpy_ref[...] * float(GRID) - 0.5 + offy
    x0f = jnp.floor(x)
    y0f = jnp.floor(y)
    fx = x - x0f
    fy = y - y0f
    lim = float(GRID - 1)
    vx0 = ((x0f >= 0.0) & (x0f <= lim)).astype(_f32)
    vx1 = ((x0f >= -1.0) & (x0f <= lim - 1.0)).astype(_f32)
    vy0 = ((y0f >= 0.0) & (y0f <= lim)).astype(_f32)
    vy1 = ((y0f >= -1.0) & (y0f <= lim - 1.0)).astype(_f32)
    base = base_ref[...]
    jq = jnp.clip(base + y0f * float(GRID) + x0f + 130.0, 0.0,
                  float(NTAB + 129))
    idx_ref[...] = jq.astype(_i32)
    wcat = jnp.concatenate(
        [attw * (1.0 - fy) * vy0 * (1.0 - fx) * vx0,
         attw * (1.0 - fy) * vy0 * fx * vx1,
         attw * fy * vy1 * (1.0 - fx) * vx0,
         attw * fy * vy1 * fx * vx1],
        axis=-1)
    wts_ref[...] = jnp.dot(wcat, perm_ref[...], preferred_element_type=_f32)


def _tc2_body(s_ref, wo_ref, bo_ref, q_ref, out_ref):
    out_ref[0] = (jnp.dot(s_ref[...], wo_ref[...], preferred_element_type=_f32)
                  + bo_ref[...] + q_ref[0])


def _sc_body(v_ref, idx_ref, w_ref, out_ref, ib, wbuf, ring0, ring1, obuf,
             sem0, sem1):
    rings = (ring0, ring1)
    sems = (sem0, sem1)
    nc = 2
    wid = lax.axis_index("s") * nc + lax.axis_index("c")
    q0 = wid * QC

    def issue(slot, sbase):
        for ql in range(SQ):
            pltpu.async_copy(v_ref.at[ib.at[sbase + ql]],
                             rings[slot].at[ql], sems[slot])

    def wait_slot(slot):
        for ql in range(SQ):
            pltpu.make_async_copy(v_ref.at[ib.at[0]],
                                  rings[slot].at[ql], sems[slot]).wait()

    def compute(slot, qloc):
        rg = rings[slot]
        for ql in range(SQ):
            def hn_body(hn, _):
                wv = wbuf[qloc + ql, pl.ds(hn * 16, 16)]
                acc0 = jnp.zeros((16,), _f32)
                acc1 = jnp.zeros((16,), _f32)
                for p in range(NP):
                    for c in range(4):
                        w = wv[p * 4 + c]
                        acc0 = acc0 + w * rg[ql, hn * 4 + p, pl.ds(c * 32, 16)]
                        acc1 = acc1 + w * rg[ql, hn * 4 + p,
                                             pl.ds(c * 32 + 16, 16)]
                obuf[ql, pl.ds(hn * D, 16)] = acc0
                obuf[ql, pl.ds(hn * D + 16, 16)] = acc1
                return 0
            lax.fori_loop(0, NH * NN, hn_body, 0)

    def sup_body(sup, _):
        qb = q0 + sup * SUP
        pltpu.sync_copy(idx_ref.at[pl.ds(qb, SUP), :], ib)
        pltpu.sync_copy(w_ref.at[pl.ds(qb, SUP), :], wbuf)
        issue(0, 0)
        issue(1, SQ)

        def pair_body(s2, _):
            s = s2 * 2
            wait_slot(0)
            compute(0, s * SQ)
            pltpu.sync_copy(obuf, out_ref.at[pl.ds(qb + s * SQ, SQ), :])
            issue(0, (s + 2) * SQ)
            wait_slot(1)
            compute(1, (s + 1) * SQ)
            pltpu.sync_copy(obuf, out_ref.at[pl.ds(qb + (s + 1) * SQ, SQ), :])
            issue(1, (s + 3) * SQ)
            return 0

        lax.fori_loop(0, NSTEP // 2 - 1, pair_body, 0)
        s = NSTEP - 2
        wait_slot(0)
        compute(0, s * SQ)
        pltpu.sync_copy(obuf, out_ref.at[pl.ds(qb + s * SQ, SQ), :])
        wait_slot(1)
        compute(1, (s + 1) * SQ)
        pltpu.sync_copy(obuf, out_ref.at[pl.ds(qb + (s + 1) * SQ, SQ), :])
        return 0

    lax.fori_loop(0, NSUP, sup_body, 0)


_sc_gather = pl.kernel(
    _sc_body,
    out_type=jax.ShapeDtypeStruct((Q, NH * NN * D), _f32),
    mesh=plsc.VectorSubcoreMesh(core_axis_name="c", subcore_axis_name="s"),
    scratch_types=(
        [pltpu.VMEM((SUP, 64), _i32),
         pltpu.VMEM((SUP, 256), _f32),
         pltpu.VMEM((SQ, 64, 128), _f32),
         pltpu.VMEM((SQ, 64, 128), _f32),
         pltpu.VMEM((SQ, NH * NN * D), _f32),
         pltpu.SemaphoreType.DMA, pltpu.SemaphoreType.DMA]),
)


def kernel(query, reference_points, value, spatial_shapes, level_start_index,
           Wv, bv, W_off, b_off, W_att, b_att, Wo, bo):
    del spatial_shapes, level_start_index
    n_map = jnp.asarray(_N_OF_LANE)
    rp = reference_points.reshape(NN, Q, 2)
    rp_t = jnp.transpose(rp, (1, 2, 0))
    rpx_b = rp_t[:, 0, :][:, n_map]
    rpy_b = rp_t[:, 1, :][:, n_map]

    Wcat = jnp.concatenate([W_off[:, 0::2], W_off[:, 1::2], W_att], axis=1)
    bcat = jnp.concatenate([b_off[0::2], b_off[1::2], b_att]).reshape(1, 192)
    gg = jnp.asarray(_GG)
    base = jnp.asarray(_BASE_LANE)
    bv2 = bv.reshape(1, CDIM)
    Wo_eff = (0.5 * jnp.repeat(Wo.reshape(NH, 1, D, CDIM), NN, axis=1)
              ).reshape(NH * NN * D, CDIM)
    bo2 = bo.reshape(1, CDIM)

    nblk = Q // BQ
    vtab, idx, wts = pl.pallas_call(
        _tc1_body,
        grid=(nblk,),
        in_specs=[
            pl.BlockSpec((1, BQ, CDIM), lambda i: (0, i, 0)),
            pl.BlockSpec((NN, BQ, CDIM), lambda i: (0, i, 0)),
            pl.BlockSpec((BQ, 64), lambda i: (i, 0)),
            pl.BlockSpec((BQ, 64), lambda i: (i, 0)),
            pl.BlockSpec((CDIM, CDIM), lambda i: (0, 0)),
            pl.BlockSpec((1, CDIM), lambda i: (0, 0)),
            pl.BlockSpec((CDIM, 192), lambda i: (0, 0)),
            pl.BlockSpec((CDIM, 192), lambda i: (0, 0)),
            pl.BlockSpec((1, 192), lambda i: (0, 0)),
            pl.BlockSpec((64, 64), lambda i: (0, 0)),
            pl.BlockSpec((1, 64), lambda i: (0, 0)),
            pl.BlockSpec((256, 256), lambda i: (0, 0)),
        ],
        out_specs=[
            pl.BlockSpec((NN, NH, BQ, D), lambda i: (0, 0, i, 0)),
            pl.BlockSpec((BQ, 64), lambda i: (i, 0)),
            pl.BlockSpec((BQ, 256), lambda i: (i, 0)),
        ],
        out_shape=[
            jax.ShapeDtypeStruct((NN, NH, Q, D), _f32),
            jax.ShapeDtypeStruct((Q, 64), _i32),
            jax.ShapeDtypeStruct((Q, 256), _f32),
        ],
    )(query, value, rpx_b, rpy_b, Wv, bv2, Wcat[:CDIM], Wcat[CDIM:], bcat,
      gg, base, jnp.asarray(_PERM))

    v_all = vtab.reshape(NTAB, D)
    vp = jnp.concatenate(
        [jnp.broadcast_to(v_all[:1], (130, D)), v_all,
         jnp.broadcast_to(v_all[NTAB - 1:], (129, D))], axis=0)
    n2 = NTAB + 130
    quad = jnp.concatenate(
        [vp[0:n2], vp[1:n2 + 1], vp[128:n2 + 128], vp[129:n2 + 129]], axis=1)
    s = _sc_gather(quad, idx, wts)

    out = pl.pallas_call(
        _tc2_body,
        grid=(nblk,),
        in_specs=[
            pl.BlockSpec((BQ, NH * NN * D), lambda i: (i, 0)),
            pl.BlockSpec((NH * NN * D, CDIM), lambda i: (0, 0)),
            pl.BlockSpec((1, CDIM), lambda i: (0, 0)),
            pl.BlockSpec((1, BQ, CDIM), lambda i: (0, i, 0)),
        ],
        out_specs=pl.BlockSpec((1, BQ, CDIM), lambda i: (0, i, 0)),
        out_shape=jax.ShapeDtypeStruct((1, Q, CDIM), _f32),
    )(s, Wo_eff, bo2, query)
    return out


# speedup vs baseline: 81.4231x; 1.4346x over previous
import functools

import numpy as np
import jax
import jax.numpy as jnp
from jax import lax
from jax.experimental import pallas as pl
from jax.experimental.pallas import tpu as pltpu
from jax.experimental.pallas import tpu_sc as plsc

Q = 16384
CDIM = 256
NH = 8
NN = 2
NP = 4
GRID = 128
D = 32
NTAB = NN * NH * Q

BQ = 1024
NW = 32
QC = Q // NW
SUP = 32
SQ = 4
NSTEP = SUP // SQ
NSUP = QC // SUP

_f32 = jnp.float32
_i32 = jnp.int32

_H_OF_LANE = np.arange(64) // 8
_N_OF_LANE = (np.arange(64) // 4) % 2
_BASE_LANE = ((_N_OF_LANE * NH + _H_OF_LANE) * Q).astype(np.float32).reshape(1, 64)
_GG = np.kron(np.eye(16, dtype=np.float32), np.ones((4, 4), np.float32))
_PERM = np.zeros((256, 256), np.float32)
for _c in range(4):
    for _hn in range(16):
        for _p in range(4):
            _PERM[_c * 64 + _hn * 4 + _p, _hn * 16 + _p * 4 + _c] = 1.0


def _tc1_body(q_ref, val_ref, rpx_ref, rpy_ref, wv_ref, bv_ref, wtop_ref,
              wbot_ref, bcat_ref, gg_ref, base_ref, perm_ref,
              vtab_ref, idx_ref, wts_ref):
    qb = q_ref[0]
    v0 = val_ref[0]
    v1 = val_ref[1]
    vv0 = jnp.dot(v0, wv_ref[...], preferred_element_type=_f32) + bv_ref[...]
    vv1 = jnp.dot(v1, wv_ref[...], preferred_element_type=_f32) + bv_ref[...]
    for h in range(NH):
        vtab_ref[0, h] = vv0[:, h * D:(h + 1) * D]
        vtab_ref[1, h] = vv1[:, h * D:(h + 1) * D]
    t = (jnp.dot(v0, wtop_ref[...], preferred_element_type=_f32)
         + jnp.dot(qb, wbot_ref[...], preferred_element_type=_f32)
         + bcat_ref[...])
    offx = t[:, 0:64]
    offy = t[:, 64:128]
    al = t[:, 128:192]
    e = jnp.exp(al - jnp.max(al, axis=-1, keepdims=True))
    den = jnp.dot(e, gg_ref[...], preferred_element_type=_f32)
    attw = e / den
    x = rpx_ref[...] * float(GRID) - 0.5 + offx
    y = rpy_ref[...] * float(GRID) - 0.5 + offy
    x0f = jnp.floor(x)
    y0f = jnp.floor(y)
    fx = x - x0f
    fy = y - y0f
    lim = float(GRID - 1)
    vx0 = ((x0f >= 0.0) & (x0f <= lim)).astype(_f32)
    vx1 = ((x0f >= -1.0) & (x0f <= lim - 1.0)).astype(_f32)
    vy0 = ((y0f >= 0.0) & (y0f <= lim)).astype(_f32)
    vy1 = ((y0f >= -1.0) & (y0f <= lim - 1.0)).astype(_f32)
    base = base_ref[...]
    jq = jnp.clip(base + y0f * float(GRID) + x0f + 130.0, 0.0,
                  float(NTAB + 129))
    idx_ref[...] = jq.astype(_i32)
    wcat = jnp.concatenate(
        [attw * (1.0 - fy) * vy0 * (1.0 - fx) * vx0,
         attw * (1.0 - fy) * vy0 * fx * vx1,
         attw * fy * vy1 * (1.0 - fx) * vx0,
         attw * fy * vy1 * fx * vx1],
        axis=-1)
    wts_ref[...] = jnp.dot(wcat, perm_ref[...], preferred_element_type=_f32)


def _tc2_body(s_ref, wo_ref, bo_ref, q_ref, out_ref):
    out_ref[0] = (jnp.dot(s_ref[...], wo_ref[...], preferred_element_type=_f32)
                  + bo_ref[...] + q_ref[0])


RQ = 1024
HALO = 136
NQUAD = ((NTAB + 130 + RQ - 1) // RQ) * RQ


def _tc3_body(v_ref, quad_ref, buf, sem):
    i = pl.program_id(0)
    nblk = pl.num_programs(0)

    @pl.when(i == 0)
    def _():
        pltpu.async_copy(v_ref.at[pl.ds(0, RQ), :],
                         buf.at[pl.ds(HALO, RQ), :], sem).wait()
        first = buf[pl.ds(HALO, 1), :]
        buf[pl.ds(0, HALO), :] = jnp.broadcast_to(first, (HALO, D))

    @pl.when((i > 0) & (i < nblk - 1))
    def _():
        o = pl.multiple_of(i * RQ - HALO, 8)
        pltpu.async_copy(v_ref.at[pl.ds(o, RQ + HALO), :], buf, sem).wait()

    @pl.when(i == nblk - 1)
    def _():
        pltpu.async_copy(v_ref.at[pl.ds(NTAB - HALO, HALO), :],
                         buf.at[pl.ds(0, HALO), :], sem).wait()
        lastrow = buf[pl.ds(HALO - 1, 1), :]
        buf[pl.ds(HALO, RQ), :] = jnp.broadcast_to(lastrow, (RQ, D))

    for c in range(4):
        off = (0, 1, 128, 129)[c]
        quad_ref[:, c * D:(c + 1) * D] = buf[pl.ds(off + 6, RQ), :]


def _sc_body(v_ref, idx_ref, w_ref, out_ref, ib0, ib1, wb0, wb1, ring0, ring1,
             ob0, ob1, gsem0, gsem1, ssem, osem0, osem1):
    rings = (ring0, ring1)
    gsems = (gsem0, gsem1)
    ibufs = (ib0, ib1)
    wbufs = (wb0, wb1)
    obufs = (ob0, ob1)
    osems = (osem0, osem1)
    nc = 2
    wid = lax.axis_index("s") * nc + lax.axis_index("c")
    q0 = wid * QC

    def stage(par, qb):
        qh = pl.multiple_of(qb // 2, SUP // 2)
        qm = pl.multiple_of(qb, SUP)
        pltpu.async_copy(idx_ref.at[pl.ds(qh, SUP // 2), :], ibufs[par],
                         ssem)
        pltpu.async_copy(w_ref.at[pl.ds(qm, SUP), :], wbufs[par], ssem)

    def stage_wait(par):
        pltpu.make_async_copy(idx_ref.at[pl.ds(0, SUP // 2), :], ibufs[par],
                              ssem).wait()
        pltpu.make_async_copy(w_ref.at[pl.ds(0, SUP), :], wbufs[par],
                              ssem).wait()

    def issue(par, slot, sbase):
        for h in range(SQ // 2):
            pltpu.async_copy(v_ref.at[ibufs[par].at[sbase // 2 + h]],
                             rings[slot].at[h], gsems[slot])

    def wait_slot(slot):
        for h in range(SQ // 2):
            pltpu.make_async_copy(v_ref.at[ib0.at[0]],
                                  rings[slot].at[h], gsems[slot]).wait()

    def compute(par, slot, qloc):
        rg = rings[slot]
        ob = obufs[slot]
        wb = wbufs[par]
        for ql in range(SQ):
            def hn_body(hn, _):
                wv = wb[qloc + ql, pl.ds(hn * 16, 16)]
                acc0 = jnp.zeros((16,), _f32)
                acc1 = jnp.zeros((16,), _f32)
                for p in range(NP):
                    for c in range(4):
                        w = wv[p * 4 + c]
                        row = (ql % 2) * 64 + hn * 4 + p
                        acc0 = acc0 + w * rg[ql // 2, row, pl.ds(c * 32, 16)]
                        acc1 = acc1 + w * rg[ql // 2, row,
                                             pl.ds(c * 32 + 16, 16)]
                ob[ql, pl.ds(hn * D, 16)] = acc0
                ob[ql, pl.ds(hn * D + 16, 16)] = acc1
                return 0
            lax.fori_loop(0, NH * NN, hn_body, 0)

    def out_push(slot, qrow):
        pltpu.async_copy(obufs[slot], out_ref.at[pl.ds(qrow, SQ), :],
                         osems[slot])

    def out_drain(slot, first):
        @pl.when(jnp.logical_not(first))
        def _():
            pltpu.make_async_copy(obufs[slot], out_ref.at[pl.ds(q0, SQ), :],
                                  osems[slot]).wait()

    def run_sup(par, sup):
        qb = q0 + sup * SUP
        first = sup == 0
        stage(1 - par, lax.min(qb + SUP, Q - SUP))

        def pair_body(s2, _):
            s = s2 * 2
            wait_slot(0)
            out_drain(0, first & (s2 == 0))
            compute(par, 0, s * SQ)
            out_push(0, qb + s * SQ)
            issue(par, 0, (s + 2) * SQ)
            wait_slot(1)
            out_drain(1, first & (s2 == 0))
            compute(par, 1, (s + 1) * SQ)
            out_push(1, qb + (s + 1) * SQ)
            issue(par, 1, (s + 3) * SQ)
            return 0

        lax.fori_loop(0, NSTEP // 2 - 1, pair_body, 0)
        s = NSTEP - 2
        last = sup == NSUP - 1
        wait_slot(0)
        out_drain(0, False)
        compute(par, 0, s * SQ)
        out_push(0, qb + s * SQ)
        stage_wait(1 - par)

        @pl.when(jnp.logical_not(last))
        def _():
            issue(1 - par, 0, 0)
        wait_slot(1)
        out_drain(1, False)
        compute(par, 1, (s + 1) * SQ)
        out_push(1, qb + (s + 1) * SQ)

        @pl.when(jnp.logical_not(last))
        def _():
            issue(1 - par, 1, SQ)

    stage(0, q0)
    stage_wait(0)
    issue(0, 0, 0)
    issue(0, 1, SQ)

    def sup2_body(s2, _):
        run_sup(0, s2 * 2)
        run_sup(1, s2 * 2 + 1)
        return 0

    lax.fori_loop(0, NSUP // 2, sup2_body, 0)
    pltpu.make_async_copy(ob0, out_ref.at[pl.ds(q0, SQ), :], osem0).wait()
    pltpu.make_async_copy(ob1, out_ref.at[pl.ds(q0, SQ), :], osem1).wait()


_sc_gather = pl.kernel(
    _sc_body,
    out_type=jax.ShapeDtypeStruct((Q, NH * NN * D), _f32),
    mesh=plsc.VectorSubcoreMesh(core_axis_name="c", subcore_axis_name="s"),
    scratch_types=(
        [pltpu.VMEM((SUP // 2, 128), _i32),
         pltpu.VMEM((SUP // 2, 128), _i32),
         pltpu.VMEM((SUP, 256), _f32), pltpu.VMEM((SUP, 256), _f32),
         pltpu.VMEM((SQ // 2, 128, 128), _f32),
         pltpu.VMEM((SQ // 2, 128, 128), _f32),
         pltpu.VMEM((SQ, NH * NN * D), _f32),
         pltpu.VMEM((SQ, NH * NN * D), _f32),
         pltpu.SemaphoreType.DMA, pltpu.SemaphoreType.DMA,
         pltpu.SemaphoreType.DMA, pltpu.SemaphoreType.DMA,
         pltpu.SemaphoreType.DMA]),
)


def kernel(query, reference_points, value, spatial_shapes, level_start_index,
           Wv, bv, W_off, b_off, W_att, b_att, Wo, bo):
    del spatial_shapes, level_start_index
    n_map = jnp.asarray(_N_OF_LANE)
    rp = reference_points.reshape(NN, Q, 2)
    rp_t = jnp.transpose(rp, (1, 2, 0))
    rpx_b = rp_t[:, 0, :][:, n_map]
    rpy_b = rp_t[:, 1, :][:, n_map]

    Wcat = jnp.concatenate([W_off[:, 0::2], W_off[:, 1::2], W_att], axis=1)
    bcat = jnp.concatenate([b_off[0::2], b_off[1::2], b_att]).reshape(1, 192)
    gg = jnp.asarray(_GG)
    base = jnp.asarray(_BASE_LANE)
    bv2 = bv.reshape(1, CDIM)
    Wo_eff = (0.5 * jnp.repeat(Wo.reshape(NH, 1, D, CDIM), NN, axis=1)
              ).reshape(NH * NN * D, CDIM)
    bo2 = bo.reshape(1, CDIM)

    nblk = Q // BQ
    vtab, idx, wts = pl.pallas_call(
        _tc1_body,
        grid=(nblk,),
        in_specs=[
            pl.BlockSpec((1, BQ, CDIM), lambda i: (0, i, 0)),
            pl.BlockSpec((NN, BQ, CDIM), lambda i: (0, i, 0)),
            pl.BlockSpec((BQ, 64), lambda i: (i, 0)),
            pl.BlockSpec((BQ, 64), lambda i: (i, 0)),
            pl.BlockSpec((CDIM, CDIM), lambda i: (0, 0)),
            pl.BlockSpec((1, CDIM), lambda i: (0, 0)),
            pl.BlockSpec((CDIM, 192), lambda i: (0, 0)),
            pl.BlockSpec((CDIM, 192), lambda i: (0, 0)),
            pl.BlockSpec((1, 192), lambda i: (0, 0)),
            pl.BlockSpec((64, 64), lambda i: (0, 0)),
            pl.BlockSpec((1, 64), lambda i: (0, 0)),
            pl.BlockSpec((256, 256), lambda i: (0, 0)),
        ],
        out_specs=[
            pl.BlockSpec((NN, NH, BQ, D), lambda i: (0, 0, i, 0)),
            pl.BlockSpec((BQ, 64), lambda i: (i, 0)),
            pl.BlockSpec((BQ, 256), lambda i: (i, 0)),
        ],
        out_shape=[
            jax.ShapeDtypeStruct((NN, NH, Q, D), _f32),
            jax.ShapeDtypeStruct((Q, 64), _i32),
            jax.ShapeDtypeStruct((Q, 256), _f32),
        ],
    )(query, value, rpx_b, rpy_b, Wv, bv2, Wcat[:CDIM], Wcat[CDIM:], bcat,
      gg, base, jnp.asarray(_PERM))

    v_all = vtab.reshape(NTAB, D)
    quad = pl.pallas_call(
        _tc3_body,
        grid=(NQUAD // RQ,),
        in_specs=[pl.BlockSpec(memory_space=pl.ANY)],
        out_specs=pl.BlockSpec((RQ, 128), lambda i: (i, 0)),
        out_shape=jax.ShapeDtypeStruct((NQUAD, 128), _f32),
        scratch_shapes=[pltpu.VMEM((RQ + HALO, D), _f32),
                        pltpu.SemaphoreType.DMA],
    )(v_all)
    s = _sc_gather(quad, idx.reshape(Q // 2, 128), wts)

    out = pl.pallas_call(
        _tc2_body,
        grid=(nblk,),
        in_specs=[
            pl.BlockSpec((BQ, NH * NN * D), lambda i: (i, 0)),
            pl.BlockSpec((NH * NN * D, CDIM), lambda i: (0, 0)),
            pl.BlockSpec((1, CDIM), lambda i: (0, 0)),
            pl.BlockSpec((1, BQ, CDIM), lambda i: (0, i, 0)),
        ],
        out_specs=pl.BlockSpec((1, BQ, CDIM), lambda i: (0, i, 0)),
        out_shape=jax.ShapeDtypeStruct((1, Q, CDIM), _f32),
    )(s, Wo_eff, bo2, query)
    return out


# speedup vs baseline: 122.1482x; 1.5002x over previous
import functools

import numpy as np
import jax
import jax.numpy as jnp
from jax import lax
from jax.experimental import pallas as pl
from jax.experimental.pallas import tpu as pltpu
from jax.experimental.pallas import tpu_sc as plsc

Q = 16384
CDIM = 256
NH = 8
NN = 2
NP = 4
GRID = 128
D = 32
NTAB = NN * NH * Q

BQ = 1024
NW = 32
QC = Q // NW
SUP = 32
SQ = 4
NSTEP = SUP // SQ
NSUP = QC // SUP

_f32 = jnp.float32
_i32 = jnp.int32

_H_OF_LANE = np.arange(64) // 8
_N_OF_LANE = (np.arange(64) // 4) % 2
_BASE_LANE = ((_N_OF_LANE * NH + _H_OF_LANE) * Q).astype(np.float32).reshape(1, 64)
_GG = np.kron(np.eye(16, dtype=np.float32), np.ones((4, 4), np.float32))
_PERM = np.zeros((256, 256), np.float32)
for _c in range(4):
    for _hn in range(16):
        for _p in range(4):
            _PERM[_c * 64 + _hn * 4 + _p, _hn * 16 + _p * 4 + _c] = 1.0


def _tc1_body(q_ref, val_ref, rpx_ref, rpy_ref, wv_ref, bv_ref, wtop_ref,
              wbot_ref, bcat_ref, gg_ref, base_ref, perm_ref,
              vtab_ref, idx_ref, wts_ref):
    qb = q_ref[0]
    v0 = val_ref[0]
    v1 = val_ref[1]
    vv0 = jnp.dot(v0, wv_ref[...], preferred_element_type=_f32) + bv_ref[...]
    vv1 = jnp.dot(v1, wv_ref[...], preferred_element_type=_f32) + bv_ref[...]
    for h in range(NH):
        vtab_ref[0, h, :, 0:D] = vv0[:, h * D:(h + 1) * D]
        vtab_ref[1, h, :, 0:D] = vv1[:, h * D:(h + 1) * D]
    t = (jnp.dot(v0, wtop_ref[...], preferred_element_type=_f32)
         + jnp.dot(qb, wbot_ref[...], preferred_element_type=_f32)
         + bcat_ref[...])
    offx = t[:, 0:64]
    offy = t[:, 64:128]
    al = t[:, 128:192]
    e = jnp.exp(al - jnp.max(al, axis=-1, keepdims=True))
    den = jnp.dot(e, gg_ref[...], preferred_element_type=_f32)
    attw = e / den
    x = rpx_ref[...] * float(GRID) - 0.5 + offx
    y = rpy_ref[...] * float(GRID) - 0.5 + offy
    x0f = jnp.floor(x)
    y0f = jnp.floor(y)
    fx = x - x0f
    fy = y - y0f
    lim = float(GRID - 1)
    vx0 = ((x0f >= 0.0) & (x0f <= lim)).astype(_f32)
    vx1 = ((x0f >= -1.0) & (x0f <= lim - 1.0)).astype(_f32)
    vy0 = ((y0f >= 0.0) & (y0f <= lim)).astype(_f32)
    vy1 = ((y0f >= -1.0) & (y0f <= lim - 1.0)).astype(_f32)
    base = base_ref[...]
    jq = jnp.clip(base + y0f * float(GRID) + x0f + 130.0, 0.0,
                  float(NTAB + 129))
    idx_ref[...] = jq.astype(_i32)
    wcat = jnp.concatenate(
        [attw * (1.0 - fy) * vy0 * (1.0 - fx) * vx0,
         attw * (1.0 - fy) * vy0 * fx * vx1,
         attw * fy * vy1 * (1.0 - fx) * vx0,
         attw * fy * vy1 * fx * vx1],
        axis=-1)
    wts_ref[...] = jnp.dot(wcat, perm_ref[...], preferred_element_type=_f32)


def _tc2_body(s_ref, wo_ref, bo_ref, q_ref, out_ref):
    out_ref[0] = (jnp.dot(s_ref[...], wo_ref[...], preferred_element_type=_f32)
                  + bo_ref[...] + q_ref[0])


RQ = 2048
NBV = NTAB // RQ
NQUAD = ((NTAB + 130 + RQ - 1) // RQ) * RQ


def _tc3_body(a_ref, quad_ref, tail):
    i = pl.program_id(0)

    @pl.when(i == 0)
    def _():
        tail[...] = a_ref[pl.ds(0, 130), :]

    for c in range(4):
        off = (0, 1, 128, 129)[c]
        lo = 130 - off
        quad_ref[0:lo, c * D:(c + 1) * D] = tail[pl.ds(130 - lo, lo), 0:D]
        quad_ref[pl.ds(lo, RQ - lo), c * D:(c + 1) * D] = \
            a_ref[pl.ds(0, RQ - lo), 0:D]
    tail[...] = a_ref[pl.ds(RQ - 130, 130), :]


def _sc_body(v_ref, idx_ref, w_ref, out_ref, ib0, ib1, wb0, wb1, ring0, ring1,
             ob0, ob1, gsem0, gsem1, ssem, osem0, osem1):
    rings = (ring0, ring1)
    gsems = (gsem0, gsem1)
    ibufs = (ib0, ib1)
    wbufs = (wb0, wb1)
    obufs = (ob0, ob1)
    osems = (osem0, osem1)
    nc = 2
    wid = lax.axis_index("s") * nc + lax.axis_index("c")
    q0 = wid * QC

    def stage(par, qb):
        qh = pl.multiple_of(qb // 2, SUP // 2)
        qm = pl.multiple_of(qb, SUP)
        pltpu.async_copy(idx_ref.at[pl.ds(qh, SUP // 2), :], ibufs[par],
                         ssem)
        pltpu.async_copy(w_ref.at[pl.ds(qm, SUP), :], wbufs[par], ssem)

    def stage_wait(par):
        pltpu.make_async_copy(idx_ref.at[pl.ds(0, SUP // 2), :], ibufs[par],
                              ssem).wait()
        pltpu.make_async_copy(w_ref.at[pl.ds(0, SUP), :], wbufs[par],
                              ssem).wait()

    def issue(par, slot, sbase):
        for h in range(SQ // 2):
            pltpu.async_copy(v_ref.at[ibufs[par].at[sbase // 2 + h]],
                             rings[slot].at[h], gsems[slot])

    def wait_slot(slot):
        for h in range(SQ // 2):
            pltpu.make_async_copy(v_ref.at[ib0.at[0]],
                                  rings[slot].at[h], gsems[slot]).wait()

    def compute(par, slot, qloc):
        rg = rings[slot]
        ob = obufs[slot]
        wb = wbufs[par]
        for ql in range(SQ):
            def hn_body(hn, _):
                wv = wb[qloc + ql, pl.ds(hn * 16, 16)]
                acc0 = jnp.zeros((16,), _f32)
                acc1 = jnp.zeros((16,), _f32)
                for p in range(NP):
                    for c in range(4):
                        w = wv[p * 4 + c]
                        row = (ql % 2) * 64 + hn * 4 + p
                        acc0 = acc0 + w * rg[ql // 2, row, pl.ds(c * 32, 16)]
                        acc1 = acc1 + w * rg[ql // 2, row,
                                             pl.ds(c * 32 + 16, 16)]
                ob[ql, pl.ds(hn * D, 16)] = acc0
                ob[ql, pl.ds(hn * D + 16, 16)] = acc1
                return 0
            lax.fori_loop(0, NH * NN, hn_body, 0)

    def out_push(slot, qrow):
        pltpu.async_copy(obufs[slot], out_ref.at[pl.ds(qrow, SQ), :],
                         osems[slot])

    def out_drain(slot, first):
        @pl.when(jnp.logical_not(first))
        def _():
            pltpu.make_async_copy(obufs[slot], out_ref.at[pl.ds(q0, SQ), :],
                                  osems[slot]).wait()

    def run_sup(par, sup):
        qb = q0 + sup * SUP
        first = sup == 0
        stage(1 - par, lax.min(qb + SUP, Q - SUP))

        def pair_body(s2, _):
            s = s2 * 2
            wait_slot(0)
            out_drain(0, first & (s2 == 0))
            compute(par, 0, s * SQ)
            out_push(0, qb + s * SQ)
            issue(par, 0, (s + 2) * SQ)
            wait_slot(1)
            out_drain(1, first & (s2 == 0))
            compute(par, 1, (s + 1) * SQ)
            out_push(1, qb + (s + 1) * SQ)
            issue(par, 1, (s + 3) * SQ)
            return 0

        lax.fori_loop(0, NSTEP // 2 - 1, pair_body, 0)
        s = NSTEP - 2
        last = sup == NSUP - 1
        wait_slot(0)
        out_drain(0, False)
        compute(par, 0, s * SQ)
        out_push(0, qb + s * SQ)
        stage_wait(1 - par)

        @pl.when(jnp.logical_not(last))
        def _():
            issue(1 - par, 0, 0)
        wait_slot(1)
        out_drain(1, False)
        compute(par, 1, (s + 1) * SQ)
        out_push(1, qb + (s + 1) * SQ)

        @pl.when(jnp.logical_not(last))
        def _():
            issue(1 - par, 1, SQ)

    stage(0, q0)
    stage_wait(0)
    issue(0, 0, 0)
    issue(0, 1, SQ)

    def sup2_body(s2, _):
        run_sup(0, s2 * 2)
        run_sup(1, s2 * 2 + 1)
        return 0

    lax.fori_loop(0, NSUP // 2, sup2_body, 0)
    pltpu.make_async_copy(ob0, out_ref.at[pl.ds(q0, SQ), :], osem0).wait()
    pltpu.make_async_copy(ob1, out_ref.at[pl.ds(q0, SQ), :], osem1).wait()


_sc_gather = pl.kernel(
    _sc_body,
    out_type=jax.ShapeDtypeStruct((Q, NH * NN * D), _f32),
    mesh=plsc.VectorSubcoreMesh(core_axis_name="c", subcore_axis_name="s"),
    scratch_types=(
        [pltpu.VMEM((SUP // 2, 128), _i32),
         pltpu.VMEM((SUP // 2, 128), _i32),
         pltpu.VMEM((SUP, 256), _f32), pltpu.VMEM((SUP, 256), _f32),
         pltpu.VMEM((SQ // 2, 128, 128), _f32),
         pltpu.VMEM((SQ // 2, 128, 128), _f32),
         pltpu.VMEM((SQ, NH * NN * D), _f32),
         pltpu.VMEM((SQ, NH * NN * D), _f32),
         pltpu.SemaphoreType.DMA, pltpu.SemaphoreType.DMA,
         pltpu.SemaphoreType.DMA, pltpu.SemaphoreType.DMA,
         pltpu.SemaphoreType.DMA]),
)


def kernel(query, reference_points, value, spatial_shapes, level_start_index,
           Wv, bv, W_off, b_off, W_att, b_att, Wo, bo):
    del spatial_shapes, level_start_index
    n_map = jnp.asarray(_N_OF_LANE)
    rp = reference_points.reshape(NN, Q, 2)
    rp_t = jnp.transpose(rp, (1, 2, 0))
    rpx_b = rp_t[:, 0, :][:, n_map]
    rpy_b = rp_t[:, 1, :][:, n_map]

    Wcat = jnp.concatenate([W_off[:, 0::2], W_off[:, 1::2], W_att], axis=1)
    bcat = jnp.concatenate([b_off[0::2], b_off[1::2], b_att]).reshape(1, 192)
    gg = jnp.asarray(_GG)
    base = jnp.asarray(_BASE_LANE)
    bv2 = bv.reshape(1, CDIM)
    Wo_eff = (0.5 * jnp.repeat(Wo.reshape(NH, 1, D, CDIM), NN, axis=1)
              ).reshape(NH * NN * D, CDIM)
    bo2 = bo.reshape(1, CDIM)

    nblk = Q // BQ
    vtab, idx, wts = pl.pallas_call(
        _tc1_body,
        grid=(nblk,),
        in_specs=[
            pl.BlockSpec((1, BQ, CDIM), lambda i: (0, i, 0)),
            pl.BlockSpec((NN, BQ, CDIM), lambda i: (0, i, 0)),
            pl.BlockSpec((BQ, 64), lambda i: (i, 0)),
            pl.BlockSpec((BQ, 64), lambda i: (i, 0)),
            pl.BlockSpec((CDIM, CDIM), lambda i: (0, 0)),
            pl.BlockSpec((1, CDIM), lambda i: (0, 0)),
            pl.BlockSpec((CDIM, 192), lambda i: (0, 0)),
            pl.BlockSpec((CDIM, 192), lambda i: (0, 0)),
            pl.BlockSpec((1, 192), lambda i: (0, 0)),
            pl.BlockSpec((64, 64), lambda i: (0, 0)),
            pl.BlockSpec((1, 64), lambda i: (0, 0)),
            pl.BlockSpec((256, 256), lambda i: (0, 0)),
        ],
        out_specs=[
            pl.BlockSpec((NN, NH, BQ, 128), lambda i: (0, 0, i, 0)),
            pl.BlockSpec((BQ, 64), lambda i: (i, 0)),
            pl.BlockSpec((BQ, 256), lambda i: (i, 0)),
        ],
        out_shape=[
            jax.ShapeDtypeStruct((NN, NH, Q, 128), _f32),
            jax.ShapeDtypeStruct((Q, 64), _i32),
            jax.ShapeDtypeStruct((Q, 256), _f32),
        ],
    )(query, value, rpx_b, rpy_b, Wv, bv2, Wcat[:CDIM], Wcat[CDIM:], bcat,
      gg, base, jnp.asarray(_PERM))

    v_all = vtab.reshape(NTAB, 128)
    quad = pl.pallas_call(
        _tc3_body,
        grid=(NQUAD // RQ,),
        in_specs=[
            pl.BlockSpec((RQ, 128), lambda i: (jnp.minimum(i, NBV - 1), 0)),
        ],
        out_specs=pl.BlockSpec((RQ, 128), lambda i: (i, 0)),
        out_shape=jax.ShapeDtypeStruct((NQUAD, 128), _f32),
        scratch_shapes=[pltpu.VMEM((130, 128), _f32)],
    )(v_all)
    s = _sc_gather(quad, idx.reshape(Q // 2, 128), wts)

    out = pl.pallas_call(
        _tc2_body,
        grid=(nblk,),
        in_specs=[
            pl.BlockSpec((BQ, NH * NN * D), lambda i: (i, 0)),
            pl.BlockSpec((NH * NN * D, CDIM), lambda i: (0, 0)),
            pl.BlockSpec((1, CDIM), lambda i: (0, 0)),
            pl.BlockSpec((1, BQ, CDIM), lambda i: (0, i, 0)),
        ],
        out_specs=pl.BlockSpec((1, BQ, CDIM), lambda i: (0, i, 0)),
        out_shape=jax.ShapeDtypeStruct((1, Q, CDIM), _f32),
    )(s, Wo_eff, bo2, query)
    return out


# speedup vs baseline: 133.2757x; 1.0911x over previous
import functools

import numpy as np
import jax
import jax.numpy as jnp
from jax import lax
from jax.experimental import pallas as pl
from jax.experimental.pallas import tpu as pltpu
from jax.experimental.pallas import tpu_sc as plsc

Q = 16384
CDIM = 256
NH = 8
NN = 2
NP = 4
GRID = 128
D = 32
NTAB = NN * NH * Q

BQ = 1024
NW = 32
QC = Q // NW
SUP = 32
SQ = 4
NSTEP = SUP // SQ
NSUP = QC // SUP

_f32 = jnp.float32
_i32 = jnp.int32

_H_OF_LANE = np.arange(64) // 8
_N_OF_LANE = (np.arange(64) // 4) % 2
_BASE_LANE = ((_N_OF_LANE * NH + _H_OF_LANE) * Q).astype(np.float32).reshape(1, 64)
_GG = np.kron(np.eye(16, dtype=np.float32), np.ones((4, 4), np.float32))
_PERM = np.zeros((256, 256), np.float32)
for _c in range(4):
    for _hn in range(16):
        for _p in range(4):
            _PERM[_c * 64 + _hn * 4 + _p, _hn * 16 + _p * 4 + _c] = 1.0


def _tc1_body(q_ref, val_ref, rpx_ref, rpy_ref, wv_ref, bv_ref, wtop_ref,
              wbot_ref, bcat_ref, gg_ref, base_ref, perm_ref,
              vtab_ref, idx_ref, wts_ref):
    qb = q_ref[0]
    v0 = val_ref[0]
    v1 = val_ref[1]
    vv0 = jnp.dot(v0, wv_ref[...], preferred_element_type=_f32) + bv_ref[...]
    vv1 = jnp.dot(v1, wv_ref[...], preferred_element_type=_f32) + bv_ref[...]
    for h in range(NH):
        vtab_ref[0, h, :, 0:D] = vv0[:, h * D:(h + 1) * D].astype(jnp.bfloat16)
        vtab_ref[1, h, :, 0:D] = vv1[:, h * D:(h + 1) * D].astype(jnp.bfloat16)
    t = (jnp.dot(v0, wtop_ref[...], preferred_element_type=_f32)
         + jnp.dot(qb, wbot_ref[...], preferred_element_type=_f32)
         + bcat_ref[...])
    offx = t[:, 0:64]
    offy = t[:, 64:128]
    al = t[:, 128:192]
    e = jnp.exp(al - jnp.max(al, axis=-1, keepdims=True))
    den = jnp.dot(e, gg_ref[...], preferred_element_type=_f32)
    attw = e / den
    x = rpx_ref[...] * float(GRID) - 0.5 + offx
    y = rpy_ref[...] * float(GRID) - 0.5 + offy
    x0f = jnp.floor(x)
    y0f = jnp.floor(y)
    fx = x - x0f
    fy = y - y0f
    lim = float(GRID - 1)
    vx0 = ((x0f >= 0.0) & (x0f <= lim)).astype(_f32)
    vx1 = ((x0f >= -1.0) & (x0f <= lim - 1.0)).astype(_f32)
    vy0 = ((y0f >= 0.0) & (y0f <= lim)).astype(_f32)
    vy1 = ((y0f >= -1.0) & (y0f <= lim - 1.0)).astype(_f32)
    base = base_ref[...]
    jq = jnp.clip(base + y0f * float(GRID) + x0f + 130.0, 0.0,
                  float(NTAB + 129))
    idx_ref[...] = jq.astype(_i32)
    wcat = jnp.concatenate(
        [attw * (1.0 - fy) * vy0 * (1.0 - fx) * vx0,
         attw * (1.0 - fy) * vy0 * fx * vx1,
         attw * fy * vy1 * (1.0 - fx) * vx0,
         attw * fy * vy1 * fx * vx1],
        axis=-1)
    wts_ref[...] = jnp.dot(wcat, perm_ref[...], preferred_element_type=_f32)


def _tc2_body(s_ref, wo_ref, bo_ref, q_ref, out_ref):
    out_ref[0] = (jnp.dot(s_ref[...], wo_ref[...], preferred_element_type=_f32)
                  + bo_ref[...] + q_ref[0])


RQ = 2048
NBV = NTAB // RQ
NQUAD = ((NTAB + 130 + RQ - 1) // RQ) * RQ


def _tc3_body(a_ref, quad_ref, tail):
    i = pl.program_id(0)

    @pl.when(i == 0)
    def _():
        tail[...] = a_ref[pl.ds(0, 130), :]

    for c in range(4):
        off = (0, 1, 128, 129)[c]
        lo = 130 - off
        quad_ref[0:lo, c * D:(c + 1) * D] = \
            tail[pl.ds(130 - lo, lo), 0:D].astype(_f32)
        quad_ref[pl.ds(lo, RQ - lo), c * D:(c + 1) * D] = \
            a_ref[pl.ds(0, RQ - lo), 0:D].astype(_f32)
    tail[...] = a_ref[pl.ds(RQ - 130, 130), :]


def _sc_body(v_ref, idx_ref, w_ref, out_ref, ib0, ib1, wb0, wb1, ring0, ring1,
             ob0, ob1, gsem0, gsem1, ssem, osem0, osem1):
    rings = (ring0, ring1)
    gsems = (gsem0, gsem1)
    ibufs = (ib0, ib1)
    wbufs = (wb0, wb1)
    obufs = (ob0, ob1)
    osems = (osem0, osem1)
    nc = 2
    wid = lax.axis_index("s") * nc + lax.axis_index("c")
    q0 = wid * QC

    def stage(par, qb):
        qh = pl.multiple_of(qb // 2, SUP // 2)
        qm = pl.multiple_of(qb, SUP)
        pltpu.async_copy(idx_ref.at[pl.ds(qh, SUP // 2), :], ibufs[par],
                         ssem)
        pltpu.async_copy(w_ref.at[pl.ds(qm, SUP), :], wbufs[par], ssem)

    def stage_wait(par):
        pltpu.make_async_copy(idx_ref.at[pl.ds(0, SUP // 2), :], ibufs[par],
                              ssem).wait()
        pltpu.make_async_copy(w_ref.at[pl.ds(0, SUP), :], wbufs[par],
                              ssem).wait()

    def issue(par, slot, sbase):
        for h in range(SQ // 2):
            pltpu.async_copy(v_ref.at[ibufs[par].at[sbase // 2 + h]],
                             rings[slot].at[h], gsems[slot])

    def wait_slot(slot):
        for h in range(SQ // 2):
            pltpu.make_async_copy(v_ref.at[ib0.at[0]],
                                  rings[slot].at[h], gsems[slot]).wait()

    def compute(par, slot, qloc):
        rg = rings[slot]
        ob = obufs[slot]
        wb = wbufs[par]
        for ql in range(SQ):
            def hn_body(hn, _):
                wv = wb[qloc + ql, pl.ds(hn * 16, 16)]
                acc0 = jnp.zeros((16,), _f32)
                acc1 = jnp.zeros((16,), _f32)
                for p in range(NP):
                    for c in range(4):
                        w = wv[p * 4 + c]
                        row = (ql % 2) * 64 + hn * 4 + p
                        acc0 = acc0 + w * rg[ql // 2, row, pl.ds(c * 32, 16)]
                        acc1 = acc1 + w * rg[ql // 2, row,
                                             pl.ds(c * 32 + 16, 16)]
                ob[ql, pl.ds(hn * D, 16)] = acc0
                ob[ql, pl.ds(hn * D + 16, 16)] = acc1
                return 0
            lax.fori_loop(0, NH * NN, hn_body, 0)

    def out_push(slot, qrow):
        pltpu.async_copy(obufs[slot], out_ref.at[pl.ds(qrow, SQ), :],
                         osems[slot])

    def out_drain(slot, first):
        @pl.when(jnp.logical_not(first))
        def _():
            pltpu.make_async_copy(obufs[slot], out_ref.at[pl.ds(q0, SQ), :],
                                  osems[slot]).wait()

    def run_sup(par, sup):
        qb = q0 + sup * SUP
        first = sup == 0
        stage(1 - par, lax.min(qb + SUP, Q - SUP))

        def pair_body(s2, _):
            s = s2 * 2
            wait_slot(0)
            out_drain(0, first & (s2 == 0))
            compute(par, 0, s * SQ)
            out_push(0, qb + s * SQ)
            issue(par, 0, (s + 2) * SQ)
            wait_slot(1)
            out_drain(1, first & (s2 == 0))
            compute(par, 1, (s + 1) * SQ)
            out_push(1, qb + (s + 1) * SQ)
            issue(par, 1, (s + 3) * SQ)
            return 0

        lax.fori_loop(0, NSTEP // 2 - 1, pair_body, 0)
        s = NSTEP - 2
        last = sup == NSUP - 1
        wait_slot(0)
        out_drain(0, False)
        compute(par, 0, s * SQ)
        out_push(0, qb + s * SQ)
        stage_wait(1 - par)

        @pl.when(jnp.logical_not(last))
        def _():
            issue(1 - par, 0, 0)
        wait_slot(1)
        out_drain(1, False)
        compute(par, 1, (s + 1) * SQ)
        out_push(1, qb + (s + 1) * SQ)

        @pl.when(jnp.logical_not(last))
        def _():
            issue(1 - par, 1, SQ)

    stage(0, q0)
    stage_wait(0)
    issue(0, 0, 0)
    issue(0, 1, SQ)

    def sup2_body(s2, _):
        run_sup(0, s2 * 2)
        run_sup(1, s2 * 2 + 1)
        return 0

    lax.fori_loop(0, NSUP // 2, sup2_body, 0)
    pltpu.make_async_copy(ob0, out_ref.at[pl.ds(q0, SQ), :], osem0).wait()
    pltpu.make_async_copy(ob1, out_ref.at[pl.ds(q0, SQ), :], osem1).wait()


_sc_gather = pl.kernel(
    _sc_body,
    out_type=jax.ShapeDtypeStruct((Q, NH * NN * D), _f32),
    mesh=plsc.VectorSubcoreMesh(core_axis_name="c", subcore_axis_name="s"),
    scratch_types=(
        [pltpu.VMEM((SUP // 2, 128), _i32),
         pltpu.VMEM((SUP // 2, 128), _i32),
         pltpu.VMEM((SUP, 256), _f32), pltpu.VMEM((SUP, 256), _f32),
         pltpu.VMEM((SQ // 2, 128, 128), _f32),
         pltpu.VMEM((SQ // 2, 128, 128), _f32),
         pltpu.VMEM((SQ, NH * NN * D), _f32),
         pltpu.VMEM((SQ, NH * NN * D), _f32),
         pltpu.SemaphoreType.DMA, pltpu.SemaphoreType.DMA,
         pltpu.SemaphoreType.DMA, pltpu.SemaphoreType.DMA,
         pltpu.SemaphoreType.DMA]),
)


def kernel(query, reference_points, value, spatial_shapes, level_start_index,
           Wv, bv, W_off, b_off, W_att, b_att, Wo, bo):
    del spatial_shapes, level_start_index
    n_map = jnp.asarray(_N_OF_LANE)
    rp = reference_points.reshape(NN, Q, 2)
    rp_t = jnp.transpose(rp, (1, 2, 0))
    rpx_b = rp_t[:, 0, :][:, n_map]
    rpy_b = rp_t[:, 1, :][:, n_map]

    Wcat = jnp.concatenate([W_off[:, 0::2], W_off[:, 1::2], W_att], axis=1)
    bcat = jnp.concatenate([b_off[0::2], b_off[1::2], b_att]).reshape(1, 192)
    gg = jnp.asarray(_GG)
    base = jnp.asarray(_BASE_LANE)
    bv2 = bv.reshape(1, CDIM)
    Wo_eff = (0.5 * jnp.repeat(Wo.reshape(NH, 1, D, CDIM), NN, axis=1)
              ).reshape(NH * NN * D, CDIM)
    bo2 = bo.reshape(1, CDIM)

    nblk = Q // BQ
    vtab, idx, wts = pl.pallas_call(
        _tc1_body,
        grid=(nblk,),
        in_specs=[
            pl.BlockSpec((1, BQ, CDIM), lambda i: (0, i, 0)),
            pl.BlockSpec((NN, BQ, CDIM), lambda i: (0, i, 0)),
            pl.BlockSpec((BQ, 64), lambda i: (i, 0)),
            pl.BlockSpec((BQ, 64), lambda i: (i, 0)),
            pl.BlockSpec((CDIM, CDIM), lambda i: (0, 0)),
            pl.BlockSpec((1, CDIM), lambda i: (0, 0)),
            pl.BlockSpec((CDIM, 192), lambda i: (0, 0)),
            pl.BlockSpec((CDIM, 192), lambda i: (0, 0)),
            pl.BlockSpec((1, 192), lambda i: (0, 0)),
            pl.BlockSpec((64, 64), lambda i: (0, 0)),
            pl.BlockSpec((1, 64), lambda i: (0, 0)),
            pl.BlockSpec((256, 256), lambda i: (0, 0)),
        ],
        out_specs=[
            pl.BlockSpec((NN, NH, BQ, 128), lambda i: (0, 0, i, 0)),
            pl.BlockSpec((BQ, 64), lambda i: (i, 0)),
            pl.BlockSpec((BQ, 256), lambda i: (i, 0)),
        ],
        out_shape=[
            jax.ShapeDtypeStruct((NN, NH, Q, 128), jnp.bfloat16),
            jax.ShapeDtypeStruct((Q, 64), _i32),
            jax.ShapeDtypeStruct((Q, 256), _f32),
        ],
    )(query, value, rpx_b, rpy_b, Wv, bv2, Wcat[:CDIM], Wcat[CDIM:], bcat,
      gg, base, jnp.asarray(_PERM))

    v_all = vtab.reshape(NTAB, 128)
    quad = pl.pallas_call(
        _tc3_body,
        grid=(NQUAD // RQ,),
        in_specs=[
            pl.BlockSpec((RQ, 128), lambda i: (jnp.minimum(i, NBV - 1), 0)),
        ],
        out_specs=pl.BlockSpec((RQ, 128), lambda i: (i, 0)),
        out_shape=jax.ShapeDtypeStruct((NQUAD, 128), _f32),
        scratch_shapes=[pltpu.VMEM((130, 128), jnp.bfloat16)],
    )(v_all)
    s = _sc_gather(quad, idx.reshape(Q // 2, 128), wts)

    out = pl.pallas_call(
        _tc2_body,
        grid=(nblk,),
        in_specs=[
            pl.BlockSpec((BQ, NH * NN * D), lambda i: (i, 0)),
            pl.BlockSpec((NH * NN * D, CDIM), lambda i: (0, 0)),
            pl.BlockSpec((1, CDIM), lambda i: (0, 0)),
            pl.BlockSpec((1, BQ, CDIM), lambda i: (0, i, 0)),
        ],
        out_specs=pl.BlockSpec((1, BQ, CDIM), lambda i: (0, i, 0)),
        out_shape=jax.ShapeDtypeStruct((1, Q, CDIM), _f32),
    )(s, Wo_eff, bo2, query)
    return out
